# baseline (speedup 1.0000x reference)
import functools
import jax
import jax.numpy as jnp
from jax import lax
from jax.experimental import pallas as pl
from jax.experimental.pallas import tpu as pltpu
from jax.experimental.pallas import tpu_sc as plsc

_T = 16
_L = 16
_NX = 3


def _make_sc_kernel(B, S, D, n_workers):
    mesh = plsc.VectorSubcoreMesh(core_axis_name="c", subcore_axis_name="s")
    stripe = S // n_workers
    cpb = stripe // _T
    n_chunks = B * cpb
    BS = B * S

    scratch = (
        [pltpu.VMEM((_T, D), jnp.float32) for _ in range(_NX)]
        + [pltpu.VMEM((stripe, D), jnp.float32),
           pltpu.VMEM((3 * D,), jnp.float32),
           pltpu.VMEM((B * stripe,), jnp.int32)]
        + [pltpu.SemaphoreType.DMA for _ in range(2 * _NX)]
    )

    @functools.partial(
        pl.kernel,
        mesh=mesh,
        out_type=jax.ShapeDtypeStruct((BS, D), jnp.float32),
        scratch_types=scratch,
    )
    def sc_k(x_hbm, lab_hbm, seg_hbm, pos_hbm, out_hbm, *bufs):
        xb = bufs[:_NX]
        pos_v = bufs[_NX]
        tab_v = bufs[_NX + 1]
        idx_all = bufs[_NX + 2]
        s_in = bufs[_NX + 3:_NX + 3 + _NX]
        s_out = bufs[_NX + 3 + _NX:]

        wid = lax.axis_index("s") * 2 + lax.axis_index("c")
        pos_base = wid * stripe
        pltpu.sync_copy(seg_hbm, tab_v)
        pltpu.sync_copy(pos_hbm.at[pl.ds(pos_base, stripe)], pos_v)
        for b in range(B):
            pltpu.sync_copy(
                lab_hbm.at[pl.ds(b * S + pos_base, stripe)],
                idx_all.at[pl.ds(b * stripe, stripe)])

        def tok_of(i):
            b, k = divmod(i, cpb)
            return b * S + pos_base + k * _T

        pend_in = {}
        pend_out = {}

        def start_in(i):
            pend_in[i] = pltpu.async_copy(
                x_hbm.at[pl.ds(tok_of(i), _T)], xb[i % _NX], s_in[i % _NX])

        def compute(i):
            x_v = xb[i % _NX]
            k = i % cpb
            labv = idx_all[pl.ds(i * _T, _L)].astype(jnp.float32)
            splats = [
                lax.gather(
                    labv,
                    jnp.full((_L, 1), r, dtype=jnp.int32),
                    dimension_numbers=lax.GatherDimensionNumbers(
                        offset_dims=(), collapsed_slice_dims=(0,),
                        start_index_map=(0,)),
                    slice_sizes=(1,),
                    mode=lax.GatherScatterMode.PROMISE_IN_BOUNDS,
                )
                for r in range(_L)
            ]
            w1 = [s * (2.0 - s) for s in splats]
            w2 = [s * (s - 1.0) * 0.5 for s in splats]

            def col_body(ci, c2):
                sl = pl.ds(ci * _L, _L)
                row0 = tab_v[pl.ds(ci * _L, _L)]
                row1 = tab_v[pl.ds(D + ci * _L, _L)]
                row2 = tab_v[pl.ds(2 * D + ci * _L, _L)]
                d1 = row1 - row0
                d2 = row2 - row0
                for r in range(_L):
                    seg = row0 + w1[r] * d1 + w2[r] * d2
                    plsc.addupdate(x_v.at[r, sl], pos_v[k * _T + r, sl] + seg)
                return c2

            lax.fori_loop(0, D // _L, col_body, 0)

        start_in(0)
        for i in range(n_chunks):
            if i + 1 < n_chunks:
                j = i - 2
                if j >= 0 and j in pend_out:
                    pend_out.pop(j).wait()
                start_in(i + 1)
            pend_in.pop(i).wait()
            compute(i)
            pend_out[i] = pltpu.async_copy(
                xb[i % _NX], out_hbm.at[pl.ds(tok_of(i), _T)], s_out[i % _NX])
        for j in sorted(pend_out):
            pend_out[j].wait()

    return sc_k


def kernel(x, segment_label, seg_table, pos_table):
    B, S, D = x.shape
    BS = B * S
    n_workers = 32

    x2 = x.reshape(BS, D)
    lab = segment_label.astype(jnp.int32).reshape(BS)
    seg_flat = seg_table.reshape(3 * D)

    sc_k = _make_sc_kernel(B, S, D, n_workers)
    out = sc_k(x2, lab, seg_flat, pos_table)
    return out.reshape(B, S, D)


# speedup vs baseline: 1.0095x; 1.0095x over previous
import functools
import jax
import jax.numpy as jnp
from jax import lax
from jax.experimental import pallas as pl
from jax.experimental.pallas import tpu as pltpu
from jax.experimental.pallas import tpu_sc as plsc

_T = 16
_L = 16
_NX = 3


def _make_sc_kernel(B, S, D, n_workers):
    mesh = plsc.VectorSubcoreMesh(core_axis_name="c", subcore_axis_name="s")
    stripe = S // n_workers
    nq = stripe // _T
    n_chunks = nq * B
    BS = B * S

    scratch = (
        [pltpu.VMEM((_T, D), jnp.float32) for _ in range(_NX)]
        + [pltpu.VMEM((3 * _T * D,), jnp.float32),
           pltpu.VMEM((_T, D), jnp.float32),
           pltpu.VMEM((3 * D,), jnp.float32),
           pltpu.VMEM((B * stripe,), jnp.int32)]
        + [pltpu.SemaphoreType.DMA for _ in range(2 * _NX + 1)]
    )

    @functools.partial(
        pl.kernel,
        mesh=mesh,
        out_type=jax.ShapeDtypeStruct((BS, D), jnp.float32),
        scratch_types=scratch,
    )
    def sc_k(x_hbm, lab_hbm, seg_hbm, pos_hbm, out_hbm, *bufs):
        xb = bufs[:_NX]
        comb_v = bufs[_NX]
        ps_v = bufs[_NX + 1]
        tab_v = bufs[_NX + 2]
        idx_all = bufs[_NX + 3]
        s_in = bufs[_NX + 4:_NX + 4 + _NX]
        s_out = bufs[_NX + 4 + _NX:_NX + 4 + 2 * _NX]
        s_pos = bufs[_NX + 4 + 2 * _NX]

        wid = lax.axis_index("s") * 2 + lax.axis_index("c")
        pos_base = wid * stripe
        pltpu.sync_copy(seg_hbm, tab_v)
        for b in range(B):
            pltpu.sync_copy(
                lab_hbm.at[pl.ds(b * S + pos_base, stripe)],
                idx_all.at[pl.ds(b * stripe, stripe)])
        pltpu.sync_copy(pos_hbm.at[pl.ds(pos_base, _T)], ps_v)

        def tok_of(i):
            q, b = divmod(i, B)
            return b * S + pos_base + q * _T

        def build_comb():
            def ci_body(ci, c2):
                sl = pl.ds(ci * _L, _L)
                t0 = tab_v[pl.ds(ci * _L, _L)]
                t1 = tab_v[pl.ds(D + ci * _L, _L)]
                t2 = tab_v[pl.ds(2 * D + ci * _L, _L)]
                for r in range(_T):
                    s = ps_v[r, sl]
                    comb_v[pl.ds(r * D + ci * _L, _L)] = s + t0
                    comb_v[pl.ds((_T + r) * D + ci * _L, _L)] = s + t1
                    comb_v[pl.ds((2 * _T + r) * D + ci * _L, _L)] = s + t2
                return c2

            lax.fori_loop(0, D // _L, ci_body, 0)

        pend_in = {}
        pend_out = {}

        def start_in(i):
            pend_in[i] = pltpu.async_copy(
                x_hbm.at[pl.ds(tok_of(i), _T)], xb[i % _NX], s_in[i % _NX])

        def compute(i):
            x_v = xb[i % _NX]
            q, b = divmod(i, B)
            labv = idx_all[pl.ds(b * stripe + q * _T, _L)]
            cbases = [labv[r] * (_T * D) + r * D for r in range(_T)]

            def col_body(ci, c2):
                sl = pl.ds(ci * _L, _L)
                for r in range(_T):
                    plsc.addupdate(
                        x_v.at[r, sl], comb_v[pl.ds(cbases[r] + ci * _L, _L)])
                return c2

            lax.fori_loop(0, D // _L, col_body, 0)

        start_in(0)
        pos_pend = None
        for i in range(n_chunks):
            q, b = divmod(i, B)
            if b == 0:
                if pos_pend is not None:
                    pos_pend.wait()
                build_comb()
                if q + 1 < nq:
                    pos_pend = pltpu.async_copy(
                        pos_hbm.at[pl.ds(pos_base + (q + 1) * _T, _T)], ps_v, s_pos)
            if i + 1 < n_chunks:
                j = i - 2
                if j >= 0 and j in pend_out:
                    pend_out.pop(j).wait()
                start_in(i + 1)
            pend_in.pop(i).wait()
            compute(i)
            pend_out[i] = pltpu.async_copy(
                xb[i % _NX], out_hbm.at[pl.ds(tok_of(i), _T)], s_out[i % _NX])
        for j in sorted(pend_out):
            pend_out[j].wait()

    return sc_k


def kernel(x, segment_label, seg_table, pos_table):
    B, S, D = x.shape
    BS = B * S
    n_workers = 32

    x2 = x.reshape(BS, D)
    lab = segment_label.astype(jnp.int32).reshape(BS)
    seg_flat = seg_table.reshape(3 * D)

    sc_k = _make_sc_kernel(B, S, D, n_workers)
    out = sc_k(x2, lab, seg_flat, pos_table)
    return out.reshape(B, S, D)


# speedup vs baseline: 4.2717x; 4.2315x over previous
import jax
import jax.numpy as jnp
from jax.experimental import pallas as pl
from jax.experimental.pallas import tpu as pltpu

_ROWS = 2048


def _body(lab_ref, seg_ref, x_ref, pos_ref, o_ref):
    l = lab_ref[0, 0, :][:, None]
    s0 = seg_ref[0, :][None, :]
    s1 = seg_ref[1, :][None, :]
    s2 = seg_ref[2, :][None, :]
    seg = jnp.where(l == 0, s0, jnp.where(l == 1, s1, s2))
    o_ref[...] = x_ref[...] + pos_ref[...] + seg


def kernel(x, segment_label, seg_table, pos_table):
    B, S, D = x.shape
    BS = B * S
    rows = _ROWS
    n_blocks = BS // rows
    blocks_per_batch = S // rows

    x2 = x.reshape(BS, D)
    lab = segment_label.astype(jnp.int32).reshape(n_blocks, 1, rows)
    seg_p = jnp.zeros((8, D), seg_table.dtype).at[:3, :].set(seg_table)

    out = pl.pallas_call(
        _body,
        grid=(blocks_per_batch, B),
        in_specs=[
            pl.BlockSpec((1, 1, rows), lambda j, b: (b * blocks_per_batch + j, 0, 0)),
            pl.BlockSpec((8, D), lambda j, b: (0, 0)),
            pl.BlockSpec((rows, D), lambda j, b: (b * blocks_per_batch + j, 0)),
            pl.BlockSpec((rows, D), lambda j, b: (j, 0)),
        ],
        out_specs=pl.BlockSpec((rows, D), lambda j, b: (b * blocks_per_batch + j, 0)),
        out_shape=jax.ShapeDtypeStruct((BS, D), x.dtype),
    )(lab, seg_p, x2, pos_table)
    return out.reshape(B, S, D)
